# Initial kernel scaffold; baseline (speedup 1.0000x reference)
import jax
import jax.numpy as jnp
from jax.experimental import pallas as pl


def kernel(x, segment_ids, W1, b1, W2, b2):
    raise NotImplementedError("write your pallas kernel here")


# speedup vs baseline: 2.7402x; 2.7402x over previous
import functools

import jax
import jax.numpy as jnp
from jax import lax
from jax.experimental import pallas as pl
from jax.experimental.pallas import tpu as pltpu
from jax.experimental.pallas import tpu_sc as plsc

N, D, G = 100000, 128, 64
H = D // 16
L = 16
C = D // L
NC, NS = 2, 16
NW = NC * NS
NG8 = N // 8
MAXCNT = 3128
SBUF = MAXCNT + L
CH = 128


def _worker_range(wid):
  base = 8 * ((wid * NG8) // NW)
  nbase = 8 * (((wid + 1) * NG8) // NW)
  return base, nbase - base


def _zero16():
  return jnp.zeros((L,), jnp.float32)


def _ninf16():
  return jnp.full((L,), -jnp.inf, jnp.float32)


def _seg_at(sbuf, li):
  return sbuf[pl.ds(li, L)][0]


def _segment_reduce(x_hbm, seg_hbm, psum_hbm, pmax_hbm, pcnt_hbm,
                    sbuf, xbuf, xbuf8, accs, accm, accc):
  wid = lax.axis_index("s") * NC + lax.axis_index("c")
  base, count = _worker_range(wid)

  pltpu.sync_copy(seg_hbm.at[pl.ds(base, MAXCNT)], sbuf.at[pl.ds(0, MAXCNT)])

  def init_body(i, _):
    accs[pl.ds(i * L, L)] = _zero16()
    accm[pl.ds(i * L, L)] = _ninf16()
    return 0
  lax.fori_loop(0, G * D // L, init_body, 0)

  def initc_body(i, _):
    accc[pl.ds(i * L, L)] = _zero16()
    return 0
  lax.fori_loop(0, G, initc_body, 0)

  def flush(cur_s, cnt, s8, m8):
    for c in range(C):
      accs[pl.ds(cur_s * D + c * L, L)] = s8[c]
      accm[pl.ds(cur_s * D + c * L, L)] = m8[c]
    accc[pl.ds(cur_s * L, L)] = jnp.broadcast_to(cnt, (L,))

  def row_step(buf, r, li, carry):
    cur_s, cnt, s8, m8 = carry
    s = _seg_at(sbuf, li)
    is_new = s != cur_s

    @pl.when(is_new)
    def _():
      flush(cur_s, cnt, s8, m8)

    cnt = jnp.where(is_new, 0.0, cnt)
    s8 = tuple(jnp.where(is_new, 0.0, v) for v in s8)
    m8 = tuple(jnp.where(is_new, -jnp.inf, v) for v in m8)

    xr = tuple(buf[pl.ds(r * D + c * L, L)] for c in range(C))
    s8 = tuple(a + b for a, b in zip(s8, xr))
    m8 = tuple(jnp.maximum(a, b) for a, b in zip(m8, xr))
    return (s, cnt + 1.0, s8, m8)

  carry0 = (_seg_at(sbuf, 0), jnp.float32(0.0),
            tuple(_zero16() for _ in range(C)),
            tuple(_ninf16() for _ in range(C)))

  full = count // CH

  def chunk_body(g, carry):
    pltpu.sync_copy(x_hbm.at[pl.ds((base + g * CH) * D, CH * D)], xbuf)
    def row_body(r, carry):
      return row_step(xbuf, r, g * CH + r, carry)
    return lax.fori_loop(0, CH, row_body, carry)

  carry = lax.fori_loop(0, full, chunk_body, carry0)

  ntail = (count - full * CH) // 8

  def tail_body(t, carry):
    off = full * CH + t * 8
    pltpu.sync_copy(x_hbm.at[pl.ds((base + off) * D, 8 * D)], xbuf8)
    def row_body(r, carry):
      return row_step(xbuf8, r, off + r, carry)
    return lax.fori_loop(0, 8, row_body, carry)

  carry = lax.fori_loop(0, ntail, tail_body, carry)

  cur_s, cnt, s8, m8 = carry
  flush(cur_s, cnt, s8, m8)

  pltpu.sync_copy(accs, psum_hbm.at[pl.ds(wid * G * D, G * D)])
  pltpu.sync_copy(accm, pmax_hbm.at[pl.ds(wid * G * D, G * D)])
  pltpu.sync_copy(accc, pcnt_hbm.at[pl.ds(wid * G * L, G * L)])


def _mlp_body(psum_ref, pmax_ref, pcnt_ref, w1_ref, b1_ref, w2_ref, b2_ref,
              att_ref):
  sums = jnp.sum(psum_ref[...], axis=0)
  maxs = jnp.max(pmax_ref[...], axis=0)
  cnts = jnp.sum(pcnt_ref[..., 0], axis=0)
  avg = sums / jnp.maximum(cnts, 1.0)[:, None]
  mx = jnp.where(jnp.isfinite(maxs), maxs, 0.0)

  def mlp(h):
    hid = jnp.maximum(jnp.dot(h, w1_ref[...]) + b1_ref[...], 0.0)
    return jnp.dot(hid, w2_ref[...]) + b2_ref[...]

  att_ref[...] = jax.nn.sigmoid(mlp(avg) + mlp(mx))


_mlp_call = pl.pallas_call(
    _mlp_body,
    out_shape=jax.ShapeDtypeStruct((G, D), jnp.float32),
)


def _apply_gate(x_hbm, seg_hbm, att_hbm, out_hbm,
                sbuf, xbuf, obuf, xbuf8, obuf8, attbuf):
  wid = lax.axis_index("s") * NC + lax.axis_index("c")
  base, count = _worker_range(wid)

  pltpu.sync_copy(seg_hbm.at[pl.ds(base, MAXCNT)], sbuf.at[pl.ds(0, MAXCNT)])
  pltpu.sync_copy(att_hbm, attbuf)

  def row_step(bufin, bufout, r, li):
    s = _seg_at(sbuf, li)
    for c in range(C):
      xr = bufin[pl.ds(r * D + c * L, L)]
      ar = attbuf[pl.ds(s * D + c * L, L)]
      bufout[pl.ds(r * D + c * L, L)] = xr * ar

  full = count // CH

  def chunk_body(g, _):
    pltpu.sync_copy(x_hbm.at[pl.ds((base + g * CH) * D, CH * D)], xbuf)
    def row_body(r, _):
      row_step(xbuf, obuf, r, g * CH + r)
      return 0
    lax.fori_loop(0, CH, row_body, 0)
    pltpu.sync_copy(obuf, out_hbm.at[pl.ds((base + g * CH) * D, CH * D)])
    return 0

  lax.fori_loop(0, full, chunk_body, 0)

  ntail = (count - full * CH) // 8

  def tail_body(t, _):
    off = full * CH + t * 8
    pltpu.sync_copy(x_hbm.at[pl.ds((base + off) * D, 8 * D)], xbuf8)
    def row_body(r, _):
      row_step(xbuf8, obuf8, r, off + r)
      return 0
    lax.fori_loop(0, 8, row_body, 0)
    pltpu.sync_copy(obuf8, out_hbm.at[pl.ds((base + off) * D, 8 * D)])
    return 0

  lax.fori_loop(0, ntail, tail_body, 0)


@functools.lru_cache(maxsize=1)
def _build():
  mesh = plsc.VectorSubcoreMesh(core_axis_name="c", subcore_axis_name="s",
                                num_cores=NC, num_subcores=NS)
  reduce_call = pl.kernel(
      _segment_reduce,
      out_type=(
          jax.ShapeDtypeStruct((NW * G * D,), jnp.float32),
          jax.ShapeDtypeStruct((NW * G * D,), jnp.float32),
          jax.ShapeDtypeStruct((NW * G * L,), jnp.float32),
      ),
      mesh=mesh,
      scratch_types=[
          pltpu.VMEM((SBUF,), jnp.int32),
          pltpu.VMEM((CH * D,), jnp.float32),
          pltpu.VMEM((8 * D,), jnp.float32),
          pltpu.VMEM((G * D,), jnp.float32),
          pltpu.VMEM((G * D,), jnp.float32),
          pltpu.VMEM((G * L,), jnp.float32),
      ],
  )
  apply_call = pl.kernel(
      _apply_gate,
      out_type=jax.ShapeDtypeStruct((N * D,), jnp.float32),
      mesh=mesh,
      scratch_types=[
          pltpu.VMEM((SBUF,), jnp.int32),
          pltpu.VMEM((CH * D,), jnp.float32),
          pltpu.VMEM((CH * D,), jnp.float32),
          pltpu.VMEM((8 * D,), jnp.float32),
          pltpu.VMEM((8 * D,), jnp.float32),
          pltpu.VMEM((G * D,), jnp.float32),
      ],
  )

  @jax.jit
  def run(x, segment_ids, W1, b1, W2, b2):
    xf = x.reshape(-1)
    psum, pmax, pcnt = reduce_call(xf, segment_ids)
    att = _mlp_call(psum.reshape(NW, G, D), pmax.reshape(NW, G, D),
                    pcnt.reshape(NW, G, L), W1, b1.reshape(1, H),
                    W2, b2.reshape(1, D))
    out = apply_call(xf, segment_ids, att.reshape(-1))
    return out.reshape(N, D)

  return run


def kernel(x, segment_ids, W1, b1, W2, b2):
  return _build()(x, segment_ids, W1, b1, W2, b2)
